# baseline (speedup 1.0000x reference)
import functools

import jax
import jax.numpy as jnp
from jax import lax
from jax.experimental import pallas as pl
from jax.experimental.pallas import tpu as pltpu
from jax.experimental.pallas import tpu_sc as plsc

N_FIELDS = 26
VOCAB = 100000
EMB = 20
B = 16384

NC, NS = 2, 16
NW = NC * NS
N_ROWS = B * N_FIELDS
ROWS_PER_W = N_ROWS // NW
CHUNK = 128
K = ROWS_PER_W // CHUNK
WIN = 2 * EMB


@functools.partial(
    pl.kernel,
    out_type=jax.ShapeDtypeStruct((N_ROWS, WIN), jnp.float32),
    mesh=plsc.VectorSubcoreMesh(core_axis_name="c", subcore_axis_name="s"),
    scratch_types=[
        pltpu.VMEM((K, CHUNK), jnp.int32),
        pltpu.VMEM((CHUNK, WIN), jnp.float32),
        pltpu.SemaphoreType.DMA,
    ],
    compiler_params=pltpu.CompilerParams(use_tc_tiling_on_sc=False),
)
def _gather_kernel(table_hbm, idx_hbm, out_hbm, idx_v, rows_v, sem):
    wid = lax.axis_index("s") * NC + lax.axis_index("c")
    base = wid * ROWS_PER_W
    pltpu.sync_copy(idx_hbm.at[wid], idx_v)

    def body(j, carry):
        pltpu.async_copy(table_hbm.at[idx_v.at[j]], rows_v, sem).wait()
        pltpu.sync_copy(rows_v, out_hbm.at[pl.ds(base + j * CHUNK, CHUNK)])
        return carry

    lax.fori_loop(0, K, body, 0, unroll=False)


def kernel(x, table):
    table_win = table.reshape(N_FIELDS * VOCAB * EMB // WIN, WIN)
    offsets = (jnp.arange(N_FIELDS, dtype=jnp.int32) * VOCAB)[None, :]
    i = (x + offsets).reshape(-1)
    idx = (i >> 1).reshape(NW, K, CHUNK)
    out40 = _gather_kernel(table_win, idx)
    odd = (i & 1).astype(bool)[:, None]
    out = jnp.where(odd, out40[:, EMB:], out40[:, :EMB])
    return out.reshape(B, N_FIELDS * EMB)


# speedup vs baseline: 1.0725x; 1.0725x over previous
import functools

import jax
import jax.numpy as jnp
from jax import lax
from jax.experimental import pallas as pl
from jax.experimental.pallas import tpu as pltpu
from jax.experimental.pallas import tpu_sc as plsc

N_FIELDS = 26
VOCAB = 100000
EMB = 20
B = 16384

NC, NS = 2, 16
NW = NC * NS
BW = B // NW
WIN = 2 * EMB
VWIN = VOCAB // 2
CHUNK = 128
NCH = BW // CHUNK


@functools.partial(
    pl.kernel,
    out_type=jax.ShapeDtypeStruct((N_FIELDS, B, WIN), jnp.float32),
    mesh=plsc.VectorSubcoreMesh(core_axis_name="c", subcore_axis_name="s"),
    scratch_types=[
        pltpu.VMEM((N_FIELDS, BW), jnp.int32),
        pltpu.VMEM((BW, WIN), jnp.float32),
        pltpu.VMEM((BW, WIN), jnp.float32),
        pltpu.SemaphoreType.DMA,
        pltpu.SemaphoreType.DMA,
    ],
    compiler_params=pltpu.CompilerParams(use_tc_tiling_on_sc=False),
)
def _gather_kernel(t3, idx_hbm, out_hbm, idx_v, rows0, rows1, sem0, sem1):
    wid = lax.axis_index("s") * NC + lax.axis_index("c")
    b0 = wid * BW
    pltpu.sync_copy(idx_hbm.at[wid], idx_v)

    def fire(f, rows, sem):
        return [
            pltpu.async_copy(
                t3.at[f].at[idx_v.at[f, pl.ds(c * CHUNK, CHUNK)]],
                rows.at[pl.ds(c * CHUNK, CHUNK)],
                sem,
            )
            for c in range(NCH)
        ]

    def drain_and_store(descs, rows, f):
        for d in descs:
            d.wait()
        pltpu.sync_copy(rows, out_hbm.at[f, pl.ds(b0, BW)])

    def body(p, carry):
        f0 = 2 * p
        d0 = fire(f0, rows0, sem0)
        d1 = fire(f0 + 1, rows1, sem1)
        drain_and_store(d0, rows0, f0)
        drain_and_store(d1, rows1, f0 + 1)
        return carry

    lax.fori_loop(0, N_FIELDS // 2, body, 0, unroll=False)


def kernel(x, table):
    t3 = table.reshape(N_FIELDS, VWIN, WIN)
    idxw = (x >> 1).reshape(NW, BW, N_FIELDS).transpose(0, 2, 1)
    out40 = _gather_kernel(t3, idxw)
    odd = (x & 1).astype(bool).T[:, :, None]
    sel = jnp.where(odd, out40[:, :, EMB:], out40[:, :, :EMB])
    return sel.transpose(1, 0, 2).reshape(B, N_FIELDS * EMB)


# speedup vs baseline: 1.1324x; 1.0558x over previous
import functools

import jax
import jax.numpy as jnp
from jax import lax
from jax.experimental import pallas as pl
from jax.experimental.pallas import tpu as pltpu
from jax.experimental.pallas import tpu_sc as plsc

N_FIELDS = 26
VOCAB = 100000
EMB = 20
B = 16384

NC, NS = 2, 16
NW = NC * NS
BW = B // NW
WIN = 2 * EMB
VWIN = VOCAB // 2
CHUNK = 128
NSB = BW // CHUNK
XPAD = 32
NUNIT = N_FIELDS * NSB
GROUPS = CHUNK * EMB // 16
OUT_D = N_FIELDS * EMB


@functools.partial(
    pl.kernel,
    out_type=jax.ShapeDtypeStruct((B, OUT_D), jnp.float32),
    mesh=plsc.VectorSubcoreMesh(core_axis_name="c", subcore_axis_name="s"),
    scratch_types=[
        pltpu.VMEM((BW, XPAD), jnp.int32),
        pltpu.VMEM((N_FIELDS, BW), jnp.int32),
        pltpu.VMEM((N_FIELDS, BW), jnp.int32),
        pltpu.VMEM((CHUNK, WIN), jnp.float32),
        pltpu.VMEM((CHUNK, WIN), jnp.float32),
        pltpu.VMEM((CHUNK, OUT_D), jnp.float32),
        pltpu.VMEM((CHUNK * EMB + 16,), jnp.int32),
        pltpu.VMEM((CHUNK * EMB + 16,), jnp.int32),
        pltpu.SemaphoreType.DMA,
        pltpu.SemaphoreType.DMA,
    ],
    compiler_params=pltpu.CompilerParams(
        use_tc_tiling_on_sc=False, needs_layout_passes=False),
)
def _emb_kernel(t3, x_hbm, out_hbm, xb, idx_v, corr_v, rows0, rows1,
                big, row_t, pw_t, sem0, sem1):
    wid = lax.axis_index("s") * NC + lax.axis_index("c")
    b0 = wid * BW
    pltpu.sync_copy(x_hbm.at[pl.ds(b0, BW)], xb)

    iota = lax.iota(jnp.int32, 16)

    def const_body(j, carry):
        row_t[pl.ds(j * EMB, 16)] = jnp.full((16,), j, jnp.int32)
        row_t[pl.ds(j * EMB + 16, 16)] = jnp.full((16,), j, jnp.int32)
        pw_t[pl.ds(j * EMB, 16)] = iota
        pw_t[pl.ds(j * EMB + 16, 16)] = iota + 16
        return carry

    lax.fori_loop(0, CHUNK, const_body, 0, unroll=False)

    def prep_f(f, carry):
        def prep_g(g, carry2):
            rows = iota + g * 16
            v = plsc.load_gather(xb, [rows, jnp.full((16,), f, jnp.int32)])
            idx_v[f, pl.ds(g * 16, 16)] = v >> 1
            corr_v[f, pl.ds(g * 16, 16)] = (v & 1) * EMB
            return carry2
        return lax.fori_loop(0, BW // 16, prep_g, carry, unroll=False)

    lax.fori_loop(0, N_FIELDS, prep_f, 0, unroll=False)

    def fire(u, rows, sem):
        f = lax.rem(u, N_FIELDS)
        sb = lax.div(u, N_FIELDS)
        return pltpu.async_copy(
            t3.at[f].at[idx_v.at[f, pl.ds(sb * CHUNK, CHUNK)]], rows, sem)

    def repack_and_flush(u, rows):
        f = lax.rem(u, N_FIELDS)
        sb = lax.div(u, N_FIELDS)
        fvec = jnp.full((16,), f, jnp.int32)
        sboff = jnp.full((16,), sb * CHUNK, jnp.int32)

        def group(g, carry2):
            rowv = row_t[pl.ds(g * 16, 16)]
            pwv = pw_t[pl.ds(g * 16, 16)]
            corr = plsc.load_gather(corr_v, [fvec, sboff + rowv])
            data = plsc.load_gather(rows, [rowv, corr + pwv])
            plsc.store_scatter(big, [rowv, pwv + f * EMB], data)
            return carry2

        lax.fori_loop(0, GROUPS, group, 0, unroll=False)

        @pl.when(f == N_FIELDS - 1)
        def _():
            pltpu.sync_copy(big, out_hbm.at[pl.ds(b0 + sb * CHUNK, CHUNK)])

    def body(k, carry):
        u0 = 2 * k
        d0 = fire(u0, rows0, sem0)
        d1 = fire(u0 + 1, rows1, sem1)
        d0.wait()
        repack_and_flush(u0, rows0)
        d1.wait()
        repack_and_flush(u0 + 1, rows1)
        return carry

    lax.fori_loop(0, NUNIT // 2, body, 0, unroll=False)


def kernel(x, table):
    t3 = table.reshape(N_FIELDS, VWIN, WIN)
    xpad = jnp.pad(x, ((0, 0), (0, XPAD - N_FIELDS)))
    return _emb_kernel(t3, xpad)
